# baseline (speedup 1.0000x reference)
import jax
import jax.numpy as jnp
from jax.experimental import pallas as pl


def _add_kernel(x_ref, e_ref, o_ref):
    o_ref[...] = x_ref[...] + e_ref[...][None, :, :]


def kernel(encoded_patches, position_embedding):
    B, P, D = encoded_patches.shape
    BB = 16
    return pl.pallas_call(
        _add_kernel,
        grid=(B // BB,),
        in_specs=[
            pl.BlockSpec((BB, P, D), lambda i: (i, 0, 0)),
            pl.BlockSpec((P, D), lambda i: (0, 0)),
        ],
        out_specs=pl.BlockSpec((BB, P, D), lambda i: (i, 0, 0)),
        out_shape=jax.ShapeDtypeStruct((B, P, D), jnp.float32),
    )(encoded_patches, position_embedding)


# speedup vs baseline: 1.3340x; 1.3340x over previous
import jax
import jax.numpy as jnp
from jax.experimental import pallas as pl


def _add_kernel(x_ref, e_ref, o_ref):
    o_ref[...] = x_ref[...] + e_ref[...]


def kernel(encoded_patches, position_embedding):
    B, P, D = encoded_patches.shape
    PD = P * D
    x2 = encoded_patches.reshape(B, PD)
    e2 = position_embedding.reshape(1, PD)
    BB = 16
    out2 = pl.pallas_call(
        _add_kernel,
        grid=(B // BB,),
        in_specs=[
            pl.BlockSpec((BB, PD), lambda i: (i, 0)),
            pl.BlockSpec((1, PD), lambda i: (0, 0)),
        ],
        out_specs=pl.BlockSpec((BB, PD), lambda i: (i, 0)),
        out_shape=jax.ShapeDtypeStruct((B, PD), jnp.float32),
    )(x2, e2)
    return out2.reshape(B, P, D)
